# baseline (speedup 1.0000x reference)
import numpy as np
import jax
import jax.numpy as jnp
from jax.experimental import pallas as pl
from jax.experimental.pallas import tpu as pltpu

_NUM_CLASSES = 81
_FG = _NUM_CLASSES - 1
_N = 5000
_K = 1000
_DET = 100
_W, _H = 1333.0, 800.0
_SCORE_T = 0.05
_NMS_T = 0.5
_CLIP = float(np.log(1000.0 / 16.0))
_CHUNK = 1000


def _scores_kernel(logits_ref, out_ref):
    x = logits_ref[...]
    m = jnp.max(x, axis=1, keepdims=True)
    e = jnp.exp(x - m)
    p = e / jnp.sum(e, axis=1, keepdims=True)
    fg = p[:, 1:]
    rowi = jax.lax.broadcasted_iota(jnp.int32, (_N, _FG), 0)
    coli = jax.lax.broadcasted_iota(jnp.int32, (_N, _FG), 1)
    fill = -1.0 - (rowi * _FG + coli).astype(jnp.float32) * 1e-5
    out_ref[...] = jnp.where(fg > _SCORE_T, fg, fill)


def _nms_kernel(cat_ref, idx_ref, scr_ref, boxes_ref, keep_ref, iou_ref):
    idx = idx_ref[...]
    box_idx = idx // _FG
    cls = idx % _FG

    rows_i = jax.lax.broadcasted_iota(jnp.int32, (_K, _CHUNK), 1)
    sel = jnp.zeros((_K, 4 + 4 * _NUM_CLASSES), dtype=jnp.float32)
    for k in range(_N // _CHUNK):
        oh = (rows_i + (k * _CHUNK) == box_idx).astype(jnp.float32)
        sel = sel + jnp.dot(oh, cat_ref[pl.ds(k * _CHUNK, _CHUNK), :],
                            preferred_element_type=jnp.float32)

    px1 = sel[:, 0:1]
    py1 = sel[:, 1:2]
    px2 = sel[:, 2:3]
    py2 = sel[:, 3:4]
    widths = px2 - px1
    heights = py2 - py1
    ctr_x = px1 + 0.5 * widths
    ctr_y = py1 + 0.5 * heights

    dr = sel[:, 4:]
    colc = jax.lax.broadcasted_iota(jnp.int32, (_K, 4 * _NUM_CLASSES), 1) // 4
    picked = dr * (colc == (cls + 1)).astype(jnp.float32)
    er = jax.lax.broadcasted_iota(jnp.int32, (4 * _NUM_CLASSES, 4), 0) % 4
    ec = jax.lax.broadcasted_iota(jnp.int32, (4 * _NUM_CLASSES, 4), 1)
    e_mat = (er == ec).astype(jnp.float32)
    d4 = jnp.dot(picked, e_mat, preferred_element_type=jnp.float32)

    dx = d4[:, 0:1] / 10.0
    dy = d4[:, 1:2] / 10.0
    dw = jnp.minimum(d4[:, 2:3] / 5.0, _CLIP)
    dh = jnp.minimum(d4[:, 3:4] / 5.0, _CLIP)
    pcx = dx * widths + ctr_x
    pcy = dy * heights + ctr_y
    pw = jnp.exp(dw) * widths
    ph = jnp.exp(dh) * heights
    x1 = jnp.clip(pcx - 0.5 * pw, 0.0, _W)
    y1 = jnp.clip(pcy - 0.5 * ph, 0.0, _H)
    x2 = jnp.clip(pcx + 0.5 * pw, 0.0, _W)
    y2 = jnp.clip(pcy + 0.5 * ph, 0.0, _H)
    boxes_ref[...] = jnp.concatenate([x1, y1, x2, y2], axis=1)

    off = cls.astype(jnp.float32) * (_W + _H)
    nx1 = x1 + off
    ny1 = y1 + off
    nx2 = x2 + off
    ny2 = y2 + off
    area = (nx2 - nx1) * (ny2 - ny1)
    nbt = jnp.transpose(
        jnp.concatenate([nx1, ny1, nx2, ny2, area], axis=1))
    rx1 = nbt[0:1, :]
    ry1 = nbt[1:2, :]
    rx2 = nbt[2:3, :]
    ry2 = nbt[3:4, :]
    rarea = nbt[4:5, :]
    iw = jnp.maximum(jnp.minimum(nx2, rx2) - jnp.maximum(nx1, rx1), 0.0)
    ih = jnp.maximum(jnp.minimum(ny2, ry2) - jnp.maximum(ny1, ry1), 0.0)
    inter = iw * ih
    iou_ref[...] = inter / (area + rarea - inter + 1e-9)

    keep0 = (scr_ref[...] > 0.0).astype(jnp.float32)
    lane = jax.lax.broadcasted_iota(jnp.int32, (1, _K), 1)

    def body(i, keep):
        row = iou_ref[pl.ds(i, 1), :]
        keep_i = jnp.sum(jnp.where(lane == i, keep, 0.0))
        sup = (row > _NMS_T) & (lane > i) & (keep_i > 0.0)
        return jnp.where(sup, 0.0, keep)

    keep_ref[...] = jax.lax.fori_loop(0, _K, body, keep0)


def kernel(class_logits, regression_outputs, proposals):
    scores_fg = pl.pallas_call(
        _scores_kernel,
        out_shape=jax.ShapeDtypeStruct((_N, _FG), jnp.float32),
    )(class_logits)

    flat = scores_fg.reshape(-1)
    av, ai = jax.lax.approx_max_k(flat, _K, recall_target=0.95)
    cnt = jnp.sum((flat > av[_K - 1]).astype(jnp.int32))

    def _exact_topk(_):
        chunks = 16
        cv, ci = jax.lax.top_k(flat.reshape(chunks, -1), _K)
        gi = ci + jnp.arange(chunks, dtype=jnp.int32)[:, None] * (
            _N * _FG // chunks)
        tv, ti = jax.lax.top_k(cv.reshape(-1), _K)
        return tv, gi.reshape(-1)[ti]

    tv, topk_idx = jax.lax.cond(
        cnt <= _K - 1, lambda _: (av, ai.astype(jnp.int32)), _exact_topk,
        operand=None)
    topk_scores = jnp.where(tv > 0.0, tv, -1.0)

    cat = jnp.concatenate([proposals, regression_outputs], axis=1)
    sel_boxes, keepf = pl.pallas_call(
        _nms_kernel,
        out_shape=(
            jax.ShapeDtypeStruct((_K, 4), jnp.float32),
            jax.ShapeDtypeStruct((1, _K), jnp.float32),
        ),
        scratch_shapes=[pltpu.VMEM((_K, _K), jnp.float32)],
    )(cat, topk_idx.reshape(_K, 1), topk_scores.reshape(1, _K))

    final_scores = jnp.where(keepf[0] > 0.0, topk_scores, -1.0)
    det_scores, det_idx = jax.lax.top_k(final_scores, _DET)
    det_boxes = sel_boxes[det_idx]
    det_labels = ((topk_idx % _FG)[det_idx] + 1).astype(jnp.float32)
    return jnp.concatenate(
        [det_boxes, det_scores[:, None], det_labels[:, None]], axis=1)


# speedup vs baseline: 1.0132x; 1.0132x over previous
import numpy as np
import jax
import jax.numpy as jnp
from jax.experimental import pallas as pl
from jax.experimental.pallas import tpu as pltpu

_NUM_CLASSES = 81
_FG = _NUM_CLASSES - 1
_N = 5000
_K = 1000
_DET = 100
_W, _H = 1333.0, 800.0
_SCORE_T = 0.05
_NMS_T = 0.5
_CLIP = float(np.log(1000.0 / 16.0))
_CHUNK = 1000


def _scores_kernel(logits_ref, out_ref):
    x = logits_ref[...]
    m = jnp.max(x, axis=1, keepdims=True)
    e = jnp.exp(x - m)
    p = e / jnp.sum(e, axis=1, keepdims=True)
    fg = p[:, 1:]
    rowi = jax.lax.broadcasted_iota(jnp.int32, (_N, _FG), 0)
    coli = jax.lax.broadcasted_iota(jnp.int32, (_N, _FG), 1)
    fill = -1.0 - (rowi * _FG + coli).astype(jnp.float32) * 1e-5
    out_ref[...] = jnp.where(fg > _SCORE_T, fg, fill)


def _nms_kernel(cat_ref, idx_ref, scr_ref, boxes_ref, keep_ref, iou_ref):
    idx = idx_ref[...]
    box_idx = idx // _FG
    cls = idx % _FG

    rows_i = jax.lax.broadcasted_iota(jnp.int32, (_K, _CHUNK), 1)
    sel = jnp.zeros((_K, 4 + 4 * _NUM_CLASSES), dtype=jnp.float32)
    for k in range(_N // _CHUNK):
        oh = (rows_i + (k * _CHUNK) == box_idx).astype(jnp.float32)
        sel = sel + jnp.dot(oh, cat_ref[pl.ds(k * _CHUNK, _CHUNK), :],
                            preferred_element_type=jnp.float32)

    px1 = sel[:, 0:1]
    py1 = sel[:, 1:2]
    px2 = sel[:, 2:3]
    py2 = sel[:, 3:4]
    widths = px2 - px1
    heights = py2 - py1
    ctr_x = px1 + 0.5 * widths
    ctr_y = py1 + 0.5 * heights

    dr = sel[:, 4:]
    colc = jax.lax.broadcasted_iota(jnp.int32, (_K, 4 * _NUM_CLASSES), 1) // 4
    picked = dr * (colc == (cls + 1)).astype(jnp.float32)
    er = jax.lax.broadcasted_iota(jnp.int32, (4 * _NUM_CLASSES, 4), 0) % 4
    ec = jax.lax.broadcasted_iota(jnp.int32, (4 * _NUM_CLASSES, 4), 1)
    e_mat = (er == ec).astype(jnp.float32)
    d4 = jnp.dot(picked, e_mat, preferred_element_type=jnp.float32)

    dx = d4[:, 0:1] / 10.0
    dy = d4[:, 1:2] / 10.0
    dw = jnp.minimum(d4[:, 2:3] / 5.0, _CLIP)
    dh = jnp.minimum(d4[:, 3:4] / 5.0, _CLIP)
    pcx = dx * widths + ctr_x
    pcy = dy * heights + ctr_y
    pw = jnp.exp(dw) * widths
    ph = jnp.exp(dh) * heights
    x1 = jnp.clip(pcx - 0.5 * pw, 0.0, _W)
    y1 = jnp.clip(pcy - 0.5 * ph, 0.0, _H)
    x2 = jnp.clip(pcx + 0.5 * pw, 0.0, _W)
    y2 = jnp.clip(pcy + 0.5 * ph, 0.0, _H)
    boxes_ref[...] = jnp.concatenate([x1, y1, x2, y2], axis=1)

    off = cls.astype(jnp.float32) * (_W + _H)
    nx1 = x1 + off
    ny1 = y1 + off
    nx2 = x2 + off
    ny2 = y2 + off
    area = (nx2 - nx1) * (ny2 - ny1)
    nbt = jnp.transpose(
        jnp.concatenate([nx1, ny1, nx2, ny2, area], axis=1))
    rx1 = nbt[0:1, :]
    ry1 = nbt[1:2, :]
    rx2 = nbt[2:3, :]
    ry2 = nbt[3:4, :]
    rarea = nbt[4:5, :]
    iw = jnp.maximum(jnp.minimum(nx2, rx2) - jnp.maximum(nx1, rx1), 0.0)
    ih = jnp.maximum(jnp.minimum(ny2, ry2) - jnp.maximum(ny1, ry1), 0.0)
    inter = iw * ih
    iou = inter / (area + rarea - inter + 1e-9)
    subi = jax.lax.broadcasted_iota(jnp.int32, (_K, _K), 0)
    lanei = jax.lax.broadcasted_iota(jnp.int32, (_K, _K), 1)
    iou_ref[...] = ((iou > _NMS_T) & (lanei > subi)).astype(jnp.float32)

    keep0 = (scr_ref[...] > 0.0).astype(jnp.float32)
    lane = jax.lax.broadcasted_iota(jnp.int32, (1, _K), 1)

    def body(i, keep):
        row = iou_ref[pl.ds(i, 1), :]
        keep_i = jnp.sum(jnp.where(lane == i, keep, 0.0))
        sup = (row > 0.0) & (keep_i > 0.0)
        return jnp.where(sup, 0.0, keep)

    keep_ref[...] = jax.lax.fori_loop(0, _K, body, keep0)


def kernel(class_logits, regression_outputs, proposals):
    scores_fg = pl.pallas_call(
        _scores_kernel,
        out_shape=jax.ShapeDtypeStruct((_N, _FG), jnp.float32),
    )(class_logits)

    chunks = 16
    cv, ci = jax.lax.top_k(scores_fg.reshape(chunks, -1), _K)
    gi = ci + jnp.arange(chunks, dtype=jnp.int32)[:, None] * (_N * _FG // chunks)
    tv, ti = jax.lax.top_k(cv.reshape(-1), _K)
    topk_idx = gi.reshape(-1)[ti]
    topk_scores = jnp.where(tv > 0.0, tv, -1.0)

    cat = jnp.concatenate([proposals, regression_outputs], axis=1)
    sel_boxes, keepf = pl.pallas_call(
        _nms_kernel,
        out_shape=(
            jax.ShapeDtypeStruct((_K, 4), jnp.float32),
            jax.ShapeDtypeStruct((1, _K), jnp.float32),
        ),
        scratch_shapes=[pltpu.VMEM((_K, _K), jnp.float32)],
    )(cat, topk_idx.reshape(_K, 1), topk_scores.reshape(1, _K))

    final_scores = jnp.where(keepf[0] > 0.0, topk_scores, -1.0)
    det_scores, det_idx = jax.lax.top_k(final_scores, _DET)
    det_boxes = sel_boxes[det_idx]
    det_labels = ((topk_idx % _FG)[det_idx] + 1).astype(jnp.float32)
    return jnp.concatenate(
        [det_boxes, det_scores[:, None], det_labels[:, None]], axis=1)
